# baseline (speedup 1.0000x reference)
import jax
import jax.numpy as jnp
from jax.experimental import pallas as pl

BL = 512


def _add_kernel(x_ref, pos_ref, o_ref):
    o_ref[...] = x_ref[...] + pos_ref[...]


def kernel(x, pos_emb_weight):
    b, l, d = x.shape
    grid = (l // BL, b)
    return pl.pallas_call(
        _add_kernel,
        grid=grid,
        in_specs=[
            pl.BlockSpec((1, BL, d), lambda i, j: (j, i, 0)),
            pl.BlockSpec((BL, d), lambda i, j: (i, 0)),
        ],
        out_specs=pl.BlockSpec((1, BL, d), lambda i, j: (j, i, 0)),
        out_shape=jax.ShapeDtypeStruct((b, l, d), x.dtype),
    )(x, pos_emb_weight)


# speedup vs baseline: 1.1549x; 1.1549x over previous
import jax
import jax.numpy as jnp
from jax.experimental import pallas as pl

BL = 2048


def _add_kernel(x_ref, pos_ref, o_ref):
    o_ref[...] = x_ref[...] + pos_ref[...]


def kernel(x, pos_emb_weight):
    b, l, d = x.shape
    grid = (l // BL, b)
    return pl.pallas_call(
        _add_kernel,
        grid=grid,
        in_specs=[
            pl.BlockSpec((1, BL, d), lambda i, j: (j, i, 0)),
            pl.BlockSpec((BL, d), lambda i, j: (i, 0)),
        ],
        out_specs=pl.BlockSpec((1, BL, d), lambda i, j: (j, i, 0)),
        out_shape=jax.ShapeDtypeStruct((b, l, d), x.dtype),
    )(x, pos_emb_weight)
